# baseline (speedup 1.0000x reference)
import jax
import jax.numpy as jnp
from jax import lax
from jax.experimental import pallas as pl

MAX_NODES = 350
K = 10
NP = 384
BI = 64


def _linear(x, w, b):
    return x @ w.T + b


def _conv(x, w, b):
    y = lax.conv_general_dilated(x, w, (1, 1), 'SAME',
                                 dimension_numbers=('NCHW', 'OIHW', 'NCHW'))
    return y + b[None, :, None, None]


def _pool(x):
    return lax.reduce_window(x, -jnp.inf, lax.max, (1, 1, 2, 2), (1, 1, 2, 2),
                             'VALID')


def _graph_kernel(xc_ref, xr_ref, yc_ref, yr_ref, w1x_ref, w1y_ref, b1_ref,
                  w2_ref, b2_ref, lw_ref, rw_ref, b1e_ref, a_ref, bm_ref):
    f32 = jnp.float32
    xc = xc_ref[0]
    xr = xr_ref[0]
    yc = yc_ref[0]
    yr = yr_ref[0]

    dx = xc - xr
    dy = yc - yr
    s = dx * dx + dy * dy
    d = jnp.sqrt(jnp.maximum(s, 1e-12))

    iota_r = lax.broadcasted_iota(jnp.int32, (NP, NP), 0)
    iota_c = lax.broadcasted_iota(jnp.int32, (NP, NP), 1)
    valid = (iota_r != iota_c) & (iota_c < MAX_NODES) & (iota_r < MAX_NODES)
    dm = jnp.where(valid, d, 1e9)

    sel_mask = jnp.zeros((NP, NP), f32)
    for _ in range(K):
        m = jnp.min(dm, axis=1, keepdims=True)
        cand = jnp.where(dm == m, iota_c, NP)
        jmin = jnp.min(cand, axis=1, keepdims=True)
        sel = iota_c == jmin
        sel_mask = sel_mask + sel.astype(f32)
        dm = jnp.where(sel, 1e9, dm)

    w_adj = sel_mask * jnp.exp(-0.5 * d)
    w_adj = jnp.where(valid, w_adj, 0.0)
    w_adj = w_adj + jnp.where((iota_r == iota_c) & (iota_r < MAX_NODES),
                              1.0, 0.0)

    ones_col = jnp.full((NP, 1), 1.0, f32)
    dn = (((0,), (0,)), ((), ()))
    deg = lax.dot_general(w_adj, ones_col, dn,
                          preferred_element_type=f32)
    dinv = jnp.where(deg > 0, jnp.maximum(deg, 1e-12) ** -0.5, 0.0)

    xl1 = xc * w1x_ref[0] + yc * w1y_ref[0]
    agg1 = lax.dot_general(w_adj, dinv * xl1, dn,
                           preferred_element_type=f32)
    h = jax.nn.relu(dinv * agg1 + b1_ref[...])

    xl2 = lax.dot_general(h, w2_ref[...], (((1,), (1,)), ((), ())),
                          preferred_element_type=f32)
    agg2 = lax.dot_general(w_adj, dinv * xl2, dn,
                           preferred_element_type=f32)
    f = jax.nn.relu(dinv * agg2 + b2_ref[...])

    row_real = lax.broadcasted_iota(jnp.int32, (NP, 1), 0) < MAX_NODES
    a = lax.dot_general(f, lw_ref[...], (((1,), (1,)), ((), ())),
                        preferred_element_type=f32)
    bm = lax.dot_general(f, rw_ref[...], (((1,), (1,)), ((), ())),
                         preferred_element_type=f32)
    a_ref[0] = jnp.where(row_real, a + b1e_ref[...], 0.0).astype(jnp.bfloat16)
    bm_ref[0] = jnp.where(row_real, bm, 0.0).astype(jnp.bfloat16)


def _edge_mlp_kernel(a_ref, b_ref, w2_ref, b2_ref, w3p_ref, b3_ref, out_ref):
    a = a_ref[0]
    bm = b_ref[0]
    e1 = jax.nn.relu(a + bm)
    e1 = e1.reshape(BI * NP, 64)
    e2 = lax.dot_general(e1, w2_ref[...], (((1,), (1,)), ((), ())),
                         preferred_element_type=jnp.float32)
    e2 = jax.nn.relu(e2 + b2_ref[...]).astype(jnp.bfloat16)
    e3 = lax.dot_general(w3p_ref[...], e2, (((1,), (1,)), ((), ())),
                         preferred_element_type=jnp.float32)
    out_ref[0, 0] = jax.nn.sigmoid(e3[0:1] + b3_ref[0, 0])


def kernel(x, node_masks, conv1_w, conv1_b, conv2_w, conv2_b, conv3_w,
           conv3_b, enc_fc_w, enc_fc_b, reg_fc1_w, reg_fc1_b, reg_fc2_w,
           reg_fc2_b, reg_coords_w, reg_coords_b, reg_count_w, reg_count_b,
           gcn1_w, gcn1_b, gcn2_w, gcn2_b, ep_fc1_w, ep_fc1_b, ep_fc2_w,
           ep_fc2_b, ep_fc3_w, ep_fc3_b):
    f32 = jnp.float32
    h = jax.nn.relu(_conv(x, conv1_w, conv1_b))
    h = _pool(h)
    h = jax.nn.relu(_conv(h, conv2_w, conv2_b))
    h = _pool(h)
    h = jax.nn.relu(_conv(h, conv3_w, conv3_b))
    h = _pool(h)
    h = h.reshape(h.shape[0], -1)
    feat = jax.nn.relu(_linear(h, enc_fc_w, enc_fc_b))
    r = jax.nn.relu(_linear(feat, reg_fc1_w, reg_fc1_b))
    r = jax.nn.relu(_linear(r, reg_fc2_w, reg_fc2_b))
    coords = _linear(r, reg_coords_w, reg_coords_b).reshape(-1, MAX_NODES, 2)
    node_count = jax.nn.sigmoid(_linear(r, reg_count_w, reg_count_b)) * MAX_NODES

    B = coords.shape[0]
    cpad = jnp.zeros((B, NP, 2), f32).at[:, :MAX_NODES, :].set(coords)
    xc = cpad[:, :, 0:1]
    xr = xc.reshape(B, 1, NP)
    yc = cpad[:, :, 1:2]
    yr = yc.reshape(B, 1, NP)

    full = lambda *dims: pl.BlockSpec(dims, lambda b: (0,) * len(dims))
    per_b = lambda *dims: pl.BlockSpec((1,) + dims, lambda b: (b, 0, 0))

    a_mat, b_mat = pl.pallas_call(
        _graph_kernel,
        grid=(B,),
        in_specs=[
            per_b(NP, 1), per_b(1, NP), per_b(NP, 1), per_b(1, NP),
            full(1, 64), full(1, 64), full(1, 64),
            full(32, 64), full(1, 32),
            full(64, 32), full(64, 32), full(1, 64),
        ],
        out_specs=[per_b(NP, 64), per_b(NP, 64)],
        out_shape=[
            jax.ShapeDtypeStruct((B, NP, 64), jnp.bfloat16),
            jax.ShapeDtypeStruct((B, NP, 64), jnp.bfloat16),
        ],
    )(xc, xr, yc, yr,
      gcn1_w[:, 0].reshape(1, 64), gcn1_w[:, 1].reshape(1, 64),
      gcn1_b.reshape(1, 64), gcn2_w, gcn2_b.reshape(1, 32),
      ep_fc1_w[:, :32], ep_fc1_w[:, 32:], ep_fc1_b.reshape(1, 64))

    a4 = a_mat.reshape(B, NP, 1, 64)
    b4 = b_mat.reshape(B, 1, NP, 64)
    n_blk = NP // BI
    w3p = jnp.zeros((8, 32), f32).at[0].set(ep_fc3_w[0])
    eflat = pl.pallas_call(
        _edge_mlp_kernel,
        grid=(B, n_blk),
        in_specs=[
            pl.BlockSpec((1, BI, 1, 64), lambda b, i: (b, i, 0, 0)),
            pl.BlockSpec((1, 1, NP, 64), lambda b, i: (b, 0, 0, 0)),
            pl.BlockSpec((32, 64), lambda b, i: (0, 0)),
            pl.BlockSpec((1, 32), lambda b, i: (0, 0)),
            pl.BlockSpec((8, 32), lambda b, i: (0, 0)),
            pl.BlockSpec((1, 1), lambda b, i: (0, 0)),
        ],
        out_specs=pl.BlockSpec((1, 1, 1, BI * NP), lambda b, i: (b, i, 0, 0)),
        out_shape=jax.ShapeDtypeStruct((B, n_blk, 1, BI * NP), f32),
    )(a4, b4, ep_fc2_w.astype(jnp.bfloat16), ep_fc2_b.reshape(1, 32), w3p.astype(jnp.bfloat16),
      ep_fc3_b.reshape(1, 1))

    e = eflat.reshape(B, NP, NP)[:, :MAX_NODES, :MAX_NODES]
    eye = jnp.eye(MAX_NODES, dtype=bool)
    e = jnp.where(eye[None], 0.0, e)
    adjacency = (e + jnp.transpose(e, (0, 2, 1))) * 0.5
    return coords, node_count, adjacency


# speedup vs baseline: 1.0343x; 1.0343x over previous
import jax
import jax.numpy as jnp
from jax import lax
from jax.experimental import pallas as pl

MAX_NODES = 350
K = 10
NP = 384
BI = 128


def _linear(x, w, b):
    return x @ w.T + b


def _conv(x, w, b):
    y = lax.conv_general_dilated(x, w, (1, 1), 'SAME',
                                 dimension_numbers=('NCHW', 'OIHW', 'NCHW'))
    return y + b[None, :, None, None]


def _pool(x):
    return lax.reduce_window(x, -jnp.inf, lax.max, (1, 1, 2, 2), (1, 1, 2, 2),
                             'VALID')


def _graph_kernel(xc_ref, xr_ref, yc_ref, yr_ref, w1x_ref, w1y_ref, b1_ref,
                  w2_ref, b2_ref, lw_ref, rw_ref, b1e_ref, a_ref, bm_ref):
    f32 = jnp.float32
    xc = xc_ref[0]
    xr = xr_ref[0]
    yc = yc_ref[0]
    yr = yr_ref[0]

    dx = xc - xr
    dy = yc - yr
    s = dx * dx + dy * dy
    d = jnp.sqrt(jnp.maximum(s, 1e-12))

    iota_r = lax.broadcasted_iota(jnp.int32, (NP, NP), 0)
    iota_c = lax.broadcasted_iota(jnp.int32, (NP, NP), 1)
    valid = (iota_r != iota_c) & (iota_c < MAX_NODES) & (iota_r < MAX_NODES)
    dm = jnp.where(valid, d, 1e9)

    sel_mask = jnp.zeros((NP, NP), f32)
    for _ in range(K):
        m = jnp.min(dm, axis=1, keepdims=True)
        cand = jnp.where(dm == m, iota_c, NP)
        jmin = jnp.min(cand, axis=1, keepdims=True)
        sel = iota_c == jmin
        sel_mask = sel_mask + sel.astype(f32)
        dm = jnp.where(sel, 1e9, dm)

    w_adj = sel_mask * jnp.exp(-0.5 * d)
    w_adj = jnp.where(valid, w_adj, 0.0)
    w_adj = w_adj + jnp.where((iota_r == iota_c) & (iota_r < MAX_NODES),
                              1.0, 0.0)

    ones_col = jnp.full((NP, 1), 1.0, f32)
    dn = (((0,), (0,)), ((), ()))
    deg = lax.dot_general(w_adj, ones_col, dn,
                          preferred_element_type=f32)
    dinv = jnp.where(deg > 0, jnp.maximum(deg, 1e-12) ** -0.5, 0.0)

    xl1 = xc * w1x_ref[0] + yc * w1y_ref[0]
    agg1 = lax.dot_general(w_adj, dinv * xl1, dn,
                           preferred_element_type=f32)
    h = jax.nn.relu(dinv * agg1 + b1_ref[...])

    xl2 = lax.dot_general(h, w2_ref[...], (((1,), (1,)), ((), ())),
                          preferred_element_type=f32)
    agg2 = lax.dot_general(w_adj, dinv * xl2, dn,
                           preferred_element_type=f32)
    f = jax.nn.relu(dinv * agg2 + b2_ref[...])

    row_real = lax.broadcasted_iota(jnp.int32, (NP, 1), 0) < MAX_NODES
    a = lax.dot_general(f, lw_ref[...], (((1,), (1,)), ((), ())),
                        preferred_element_type=f32)
    bm = lax.dot_general(f, rw_ref[...], (((1,), (1,)), ((), ())),
                         preferred_element_type=f32)
    a_ref[0] = jnp.where(row_real, a + b1e_ref[...], 0.0)
    bm_ref[0] = jnp.where(row_real, bm, 0.0)


def _edge_mlp_kernel(a_ref, b_ref, w2_ref, b2_ref, w3p_ref, b3_ref, out_ref):
    a = a_ref[0]
    bm = b_ref[0]
    e1 = jax.nn.relu(a + bm)
    e1 = e1.reshape(BI * NP, 64)
    e2 = lax.dot_general(e1, w2_ref[...], (((1,), (1,)), ((), ())),
                         preferred_element_type=jnp.float32)
    e2 = jax.nn.relu(e2 + b2_ref[...])
    e3 = lax.dot_general(w3p_ref[...], e2, (((1,), (1,)), ((), ())),
                         preferred_element_type=jnp.float32)
    out_ref[0, 0] = jax.nn.sigmoid(e3[0:1] + b3_ref[0, 0])


def kernel(x, node_masks, conv1_w, conv1_b, conv2_w, conv2_b, conv3_w,
           conv3_b, enc_fc_w, enc_fc_b, reg_fc1_w, reg_fc1_b, reg_fc2_w,
           reg_fc2_b, reg_coords_w, reg_coords_b, reg_count_w, reg_count_b,
           gcn1_w, gcn1_b, gcn2_w, gcn2_b, ep_fc1_w, ep_fc1_b, ep_fc2_w,
           ep_fc2_b, ep_fc3_w, ep_fc3_b):
    f32 = jnp.float32
    h = jax.nn.relu(_conv(x, conv1_w, conv1_b))
    h = _pool(h)
    h = jax.nn.relu(_conv(h, conv2_w, conv2_b))
    h = _pool(h)
    h = jax.nn.relu(_conv(h, conv3_w, conv3_b))
    h = _pool(h)
    h = h.reshape(h.shape[0], -1)
    feat = jax.nn.relu(_linear(h, enc_fc_w, enc_fc_b))
    r = jax.nn.relu(_linear(feat, reg_fc1_w, reg_fc1_b))
    r = jax.nn.relu(_linear(r, reg_fc2_w, reg_fc2_b))
    coords = _linear(r, reg_coords_w, reg_coords_b).reshape(-1, MAX_NODES, 2)
    node_count = jax.nn.sigmoid(_linear(r, reg_count_w, reg_count_b)) * MAX_NODES

    B = coords.shape[0]
    cpad = jnp.zeros((B, NP, 2), f32).at[:, :MAX_NODES, :].set(coords)
    xc = cpad[:, :, 0:1]
    xr = xc.reshape(B, 1, NP)
    yc = cpad[:, :, 1:2]
    yr = yc.reshape(B, 1, NP)

    full = lambda *dims: pl.BlockSpec(dims, lambda b: (0,) * len(dims))
    per_b = lambda *dims: pl.BlockSpec((1,) + dims, lambda b: (b, 0, 0))

    a_mat, b_mat = pl.pallas_call(
        _graph_kernel,
        grid=(B,),
        in_specs=[
            per_b(NP, 1), per_b(1, NP), per_b(NP, 1), per_b(1, NP),
            full(1, 64), full(1, 64), full(1, 64),
            full(32, 64), full(1, 32),
            full(64, 32), full(64, 32), full(1, 64),
        ],
        out_specs=[per_b(NP, 64), per_b(NP, 64)],
        out_shape=[
            jax.ShapeDtypeStruct((B, NP, 64), f32),
            jax.ShapeDtypeStruct((B, NP, 64), f32),
        ],
    )(xc, xr, yc, yr,
      gcn1_w[:, 0].reshape(1, 64), gcn1_w[:, 1].reshape(1, 64),
      gcn1_b.reshape(1, 64), gcn2_w, gcn2_b.reshape(1, 32),
      ep_fc1_w[:, :32], ep_fc1_w[:, 32:], ep_fc1_b.reshape(1, 64))

    a4 = a_mat.reshape(B, NP, 1, 64)
    b4 = b_mat.reshape(B, 1, NP, 64)
    n_blk = NP // BI
    w3p = jnp.zeros((8, 32), f32).at[0].set(ep_fc3_w[0])
    eflat = pl.pallas_call(
        _edge_mlp_kernel,
        grid=(B, n_blk),
        in_specs=[
            pl.BlockSpec((1, BI, 1, 64), lambda b, i: (b, i, 0, 0)),
            pl.BlockSpec((1, 1, NP, 64), lambda b, i: (b, 0, 0, 0)),
            pl.BlockSpec((32, 64), lambda b, i: (0, 0)),
            pl.BlockSpec((1, 32), lambda b, i: (0, 0)),
            pl.BlockSpec((8, 32), lambda b, i: (0, 0)),
            pl.BlockSpec((1, 1), lambda b, i: (0, 0)),
        ],
        out_specs=pl.BlockSpec((1, 1, 1, BI * NP), lambda b, i: (b, i, 0, 0)),
        out_shape=jax.ShapeDtypeStruct((B, n_blk, 1, BI * NP), f32),
    )(a4, b4, ep_fc2_w, ep_fc2_b.reshape(1, 32), w3p,
      ep_fc3_b.reshape(1, 1))

    e = eflat.reshape(B, NP, NP)[:, :MAX_NODES, :MAX_NODES]
    eye = jnp.eye(MAX_NODES, dtype=bool)
    e = jnp.where(eye[None], 0.0, e)
    adjacency = (e + jnp.transpose(e, (0, 2, 1))) * 0.5
    return coords, node_count, adjacency


# speedup vs baseline: 1.0461x; 1.0114x over previous
import jax
import jax.numpy as jnp
from jax import lax
from jax.experimental import pallas as pl
from jax.experimental.pallas import tpu as pltpu

MAX_NODES = 350
K = 10
NP = 384
BI = 128


def _linear(x, w, b):
    return x @ w.T + b


def _conv(x, w, b):
    y = lax.conv_general_dilated(x, w, (1, 1), 'SAME',
                                 dimension_numbers=('NCHW', 'OIHW', 'NCHW'))
    return y + b[None, :, None, None]


def _pool(x):
    return lax.reduce_window(x, -jnp.inf, lax.max, (1, 1, 2, 2), (1, 1, 2, 2),
                             'VALID')


def _build_graph(xc, xr, yc, yr, w1x, w1y, b1, w2, b2, lw, rw, b1e):
    f32 = jnp.float32
    dx = xc - xr
    dy = yc - yr
    s = dx * dx + dy * dy
    d = jnp.sqrt(jnp.maximum(s, 1e-12))

    iota_r = lax.broadcasted_iota(jnp.int32, (NP, NP), 0)
    iota_c = lax.broadcasted_iota(jnp.int32, (NP, NP), 1)
    valid = (iota_r != iota_c) & (iota_c < MAX_NODES) & (iota_r < MAX_NODES)
    dm = jnp.where(valid, d, 1e9)

    sel_mask = jnp.zeros((NP, NP), f32)
    for _ in range(K):
        m = jnp.min(dm, axis=1, keepdims=True)
        cand = jnp.where(dm == m, iota_c, NP)
        jmin = jnp.min(cand, axis=1, keepdims=True)
        sel = iota_c == jmin
        sel_mask = sel_mask + sel.astype(f32)
        dm = jnp.where(sel, 1e9, dm)

    w_adj = sel_mask * jnp.exp(-0.5 * d)
    w_adj = jnp.where(valid, w_adj, 0.0)
    w_adj = w_adj + jnp.where((iota_r == iota_c) & (iota_r < MAX_NODES),
                              1.0, 0.0)

    ones_col = jnp.full((NP, 1), 1.0, f32)
    dn = (((0,), (0,)), ((), ()))
    deg = lax.dot_general(w_adj, ones_col, dn,
                          preferred_element_type=f32)
    dinv = jnp.where(deg > 0, jnp.maximum(deg, 1e-12) ** -0.5, 0.0)

    xl1 = xc * w1x + yc * w1y
    agg1 = lax.dot_general(w_adj, dinv * xl1, dn,
                           preferred_element_type=f32)
    h = jax.nn.relu(dinv * agg1 + b1)

    xl2 = lax.dot_general(h, w2, (((1,), (1,)), ((), ())),
                          preferred_element_type=f32)
    agg2 = lax.dot_general(w_adj, dinv * xl2, dn,
                           preferred_element_type=f32)
    f = jax.nn.relu(dinv * agg2 + b2)

    row_real = lax.broadcasted_iota(jnp.int32, (NP, 1), 0) < MAX_NODES
    a = lax.dot_general(f, lw, (((1,), (1,)), ((), ())),
                        preferred_element_type=f32)
    bm = lax.dot_general(f, rw, (((1,), (1,)), ((), ())),
                         preferred_element_type=f32)
    a = jnp.where(row_real, a + b1e, 0.0)
    bm = jnp.where(row_real, bm, 0.0)
    return a, bm


def _fused_kernel(xc_ref, xr_ref, yc_ref, yr_ref, w1x_ref, w1y_ref, b1_ref,
                  w2_ref, b2_ref, lw_ref, rw_ref, b1e_ref, w2e_ref, b2e_ref,
                  w3p_ref, b3_ref, out_ref, a_scr, bm_scr):
    i_blk = pl.program_id(1)

    @pl.when(i_blk == 0)
    def _():
        a, bm = _build_graph(xc_ref[0], xr_ref[0], yc_ref[0], yr_ref[0],
                             w1x_ref[0], w1y_ref[0], b1_ref[...],
                             w2_ref[...], b2_ref[...], lw_ref[...],
                             rw_ref[...], b1e_ref[...])
        a_scr[...] = a
        bm_scr[...] = bm

    a = a_scr[pl.ds(i_blk * BI, BI), :].reshape(BI, 1, 64)
    bm = bm_scr[...].reshape(1, NP, 64)
    e1 = jax.nn.relu(a + bm)
    e1 = e1.reshape(BI * NP, 64)
    e2 = lax.dot_general(e1, w2e_ref[...], (((1,), (1,)), ((), ())),
                         preferred_element_type=jnp.float32)
    e2 = jax.nn.relu(e2 + b2e_ref[...])
    e3 = lax.dot_general(w3p_ref[...], e2, (((1,), (1,)), ((), ())),
                         preferred_element_type=jnp.float32)
    out_ref[0, 0] = jax.nn.sigmoid(e3[0:1] + b3_ref[0, 0])


def kernel(x, node_masks, conv1_w, conv1_b, conv2_w, conv2_b, conv3_w,
           conv3_b, enc_fc_w, enc_fc_b, reg_fc1_w, reg_fc1_b, reg_fc2_w,
           reg_fc2_b, reg_coords_w, reg_coords_b, reg_count_w, reg_count_b,
           gcn1_w, gcn1_b, gcn2_w, gcn2_b, ep_fc1_w, ep_fc1_b, ep_fc2_w,
           ep_fc2_b, ep_fc3_w, ep_fc3_b):
    f32 = jnp.float32
    h = jax.nn.relu(_conv(x, conv1_w, conv1_b))
    h = _pool(h)
    h = jax.nn.relu(_conv(h, conv2_w, conv2_b))
    h = _pool(h)
    h = jax.nn.relu(_conv(h, conv3_w, conv3_b))
    h = _pool(h)
    h = h.reshape(h.shape[0], -1)
    feat = jax.nn.relu(_linear(h, enc_fc_w, enc_fc_b))
    r = jax.nn.relu(_linear(feat, reg_fc1_w, reg_fc1_b))
    r = jax.nn.relu(_linear(r, reg_fc2_w, reg_fc2_b))
    coords = _linear(r, reg_coords_w, reg_coords_b).reshape(-1, MAX_NODES, 2)
    node_count = jax.nn.sigmoid(_linear(r, reg_count_w, reg_count_b)) * MAX_NODES

    B = coords.shape[0]
    cpad = jnp.zeros((B, NP, 2), f32).at[:, :MAX_NODES, :].set(coords)
    xc = cpad[:, :, 0:1]
    xr = xc.reshape(B, 1, NP)
    yc = cpad[:, :, 1:2]
    yr = yc.reshape(B, 1, NP)

    n_blk = NP // BI
    w3p = jnp.zeros((8, 32), f32).at[0].set(ep_fc3_w[0])

    full = lambda *dims: pl.BlockSpec(dims, lambda b, i: (0,) * len(dims))
    per_b = lambda *dims: pl.BlockSpec((1,) + dims, lambda b, i: (b, 0, 0))

    eflat = pl.pallas_call(
        _fused_kernel,
        grid=(B, n_blk),
        in_specs=[
            per_b(NP, 1), per_b(1, NP), per_b(NP, 1), per_b(1, NP),
            full(1, 64), full(1, 64), full(1, 64),
            full(32, 64), full(1, 32),
            full(64, 32), full(64, 32), full(1, 64),
            full(32, 64), full(1, 32), full(8, 32), full(1, 1),
        ],
        out_specs=pl.BlockSpec((1, 1, 1, BI * NP), lambda b, i: (b, i, 0, 0)),
        out_shape=jax.ShapeDtypeStruct((B, n_blk, 1, BI * NP), f32),
        scratch_shapes=[
            pltpu.VMEM((NP, 64), f32),
            pltpu.VMEM((NP, 64), f32),
        ],
    )(xc, xr, yc, yr,
      gcn1_w[:, 0].reshape(1, 64), gcn1_w[:, 1].reshape(1, 64),
      gcn1_b.reshape(1, 64), gcn2_w, gcn2_b.reshape(1, 32),
      ep_fc1_w[:, :32], ep_fc1_w[:, 32:], ep_fc1_b.reshape(1, 64),
      ep_fc2_w, ep_fc2_b.reshape(1, 32), w3p, ep_fc3_b.reshape(1, 1))

    e = eflat.reshape(B, NP, NP)
    iota = jnp.arange(NP)
    eye = (iota[:, None] == iota[None, :])
    e = jnp.where(eye[None], 0.0, e)
    sym = (e + jnp.transpose(e, (0, 2, 1))) * 0.5
    adjacency = sym[:, :MAX_NODES, :MAX_NODES]
    return coords, node_count, adjacency
